# baseline (speedup 1.0000x reference)
import jax
import jax.numpy as jnp
from jax.experimental import pallas as pl

S = 2048
B = 2
H = 768
NH = 12
HPH = 64
TQ = 256
NT = S // TQ
HB = 4 * HPH
HE = NH * HB
SCALE = 1.0 / (HPH ** 0.5)
NEG = -1e9


def _qkv_kernel(x_ref, w_ref, b_ref, o_ref):
    o_ref[...] = (
        jnp.dot(x_ref[...], w_ref[...], preferred_element_type=jnp.float32)
        + b_ref[...]
    ).astype(jnp.bfloat16)


def _attn_kernel(kv_ref, mask_ref, wd_ref, bd_ref, o_ref):
    i = pl.program_id(1)
    bias = (mask_ref[0].astype(jnp.float32) - 1.0) * -NEG
    outs = []
    for h in range(NH):
        base = h * HB
        q = kv_ref[pl.ds(i * TQ, TQ), base:base + HPH] * jnp.bfloat16(SCALE)
        k = kv_ref[:, base + HPH:base + 2 * HPH]
        vext = kv_ref[:, base + 2 * HPH:base + HB]
        s = jax.lax.dot_general(
            q, k, (((1,), (1,)), ((), ())), preferred_element_type=jnp.float32
        )
        p = jnp.exp(s + bias).astype(jnp.bfloat16)
        oe = jax.lax.dot_general(
            p, vext, (((1,), (0,)), ((), ())),
            preferred_element_type=jnp.float32,
        )
        l = oe[:, HPH:HPH + 1]
        outs.append(oe[:, :HPH] / jnp.where(l == 0.0, 1.0, l))
    rep = jnp.concatenate(outs, axis=1).astype(jnp.bfloat16)
    o_ref[...] = (
        jnp.dot(rep, wd_ref[...], preferred_element_type=jnp.float32)
        + bd_ref[...]
    )


def kernel(hidden_states, attention_mask, W_qkv, b_qkv, W_dense, b_dense):
    x = jnp.transpose(hidden_states.astype(jnp.bfloat16), (1, 0, 2)).reshape(B * S, H)
    mask8 = (attention_mask.reshape(B, S, S) != 0).astype(jnp.int8)

    w4 = W_qkv.reshape(H, NH, 3, HPH)
    w_ext = jnp.concatenate(
        [w4[:, :, 0], w4[:, :, 1], w4[:, :, 2],
         jnp.zeros((H, NH, HPH), W_qkv.dtype)], axis=-1,
    ).reshape(H, HE).astype(jnp.bfloat16)
    b4 = b_qkv.reshape(NH, 3, HPH)
    ones_col = jnp.zeros((NH, HPH), b_qkv.dtype).at[:, 0].set(1.0)
    b_ext = jnp.concatenate(
        [b4[:, 0], b4[:, 1], b4[:, 2], ones_col], axis=-1,
    ).reshape(1, HE)
    w_dense16 = W_dense.astype(jnp.bfloat16)

    mixed = pl.pallas_call(
        _qkv_kernel,
        grid=(B * NT,),
        in_specs=[
            pl.BlockSpec((TQ, H), lambda i: (i, 0)),
            pl.BlockSpec((H, HE), lambda i: (0, 0)),
            pl.BlockSpec((1, HE), lambda i: (0, 0)),
        ],
        out_specs=pl.BlockSpec((TQ, HE), lambda i: (i, 0)),
        out_shape=jax.ShapeDtypeStruct((B * S, HE), jnp.bfloat16),
    )(x, w_ext, b_ext)

    out2 = pl.pallas_call(
        _attn_kernel,
        grid=(B, NT),
        in_specs=[
            pl.BlockSpec((S, HE), lambda b, i: (b, 0)),
            pl.BlockSpec((1, TQ, S), lambda b, i: (b, i, 0)),
            pl.BlockSpec((H, H), lambda b, i: (0, 0)),
            pl.BlockSpec((1, H), lambda b, i: (0, 0)),
        ],
        out_specs=pl.BlockSpec((TQ, H), lambda b, i: (b * NT + i, 0)),
        out_shape=jax.ShapeDtypeStruct((B * S, H), jnp.float32),
    )(mixed, mask8, w_dense16, b_dense.reshape(1, H))

    return out2.reshape(B, S, H).transpose(1, 0, 2)


# speedup vs baseline: 1.1328x; 1.1328x over previous
import jax
import jax.numpy as jnp
from jax.experimental import pallas as pl

S = 2048
B = 2
H = 768
NH = 12
HPH = 64
TQ = 256
NT = S // TQ
HB = 4 * HPH
HE = NH * HB
SCALE = 1.0 / (HPH ** 0.5)
NEG = -1e9


def _qkv_kernel(x_ref, w_ref, b_ref, o_ref):
    b = pl.program_id(0)
    x = x_ref[:, pl.ds(b * H, H)].astype(jnp.bfloat16)
    o_ref[...] = (
        jnp.dot(x, w_ref[...], preferred_element_type=jnp.float32)
        + b_ref[...]
    ).astype(jnp.bfloat16)


def _attn_kernel(kv_ref, mask_ref, wd_ref, bd_ref, o_ref):
    i = pl.program_id(1)
    bias = (mask_ref[0] - 1.0) * -NEG
    outs = []
    for h in range(NH):
        base = h * HB
        k = kv_ref[:, base:base + HPH]
        q = kv_ref[pl.ds(i * TQ, TQ), base + HPH:base + 2 * HPH] * jnp.bfloat16(SCALE)
        vext = kv_ref[:, base + 2 * HPH:base + HB]
        s = jax.lax.dot_general(
            q, k, (((1,), (1,)), ((), ())), preferred_element_type=jnp.float32
        )
        p = jnp.exp(s + bias).astype(jnp.bfloat16)
        oe = jax.lax.dot_general(
            p, vext, (((1,), (0,)), ((), ())),
            preferred_element_type=jnp.float32,
        )
        l = oe[:, HPH:HPH + 1]
        outs.append(oe[:, :HPH] / jnp.where(l == 0.0, 1.0, l))
    rep = jnp.concatenate(outs, axis=1).astype(jnp.bfloat16)
    o_ref[...] = (
        jnp.dot(rep, wd_ref[...], preferred_element_type=jnp.float32)
        + bd_ref[...]
    )


def kernel(hidden_states, attention_mask, W_qkv, b_qkv, W_dense, b_dense):
    hs2 = hidden_states.reshape(S, B * H)
    mask3 = attention_mask.reshape(B, S, S)

    w4 = W_qkv.reshape(H, NH, 3, HPH)
    w_ext = jnp.concatenate(
        [w4[:, :, 1], w4[:, :, 0], w4[:, :, 2],
         jnp.zeros((H, NH, HPH), W_qkv.dtype)], axis=-1,
    ).reshape(H, HE).astype(jnp.bfloat16)
    b4 = b_qkv.reshape(NH, 3, HPH)
    ones_col = jnp.zeros((NH, HPH), b_qkv.dtype).at[:, 0].set(1.0)
    b_ext = jnp.concatenate(
        [b4[:, 1], b4[:, 0], b4[:, 2], ones_col], axis=-1,
    ).reshape(1, HE)
    w_dense16 = W_dense.astype(jnp.bfloat16)

    mixed = pl.pallas_call(
        _qkv_kernel,
        grid=(B, NT),
        in_specs=[
            pl.BlockSpec((TQ, B * H), lambda b, i: (i, 0)),
            pl.BlockSpec((H, HE), lambda b, i: (0, 0)),
            pl.BlockSpec((1, HE), lambda b, i: (0, 0)),
        ],
        out_specs=pl.BlockSpec((TQ, HE), lambda b, i: (b * NT + i, 0)),
        out_shape=jax.ShapeDtypeStruct((B * S, HE), jnp.bfloat16),
    )(hs2, w_ext, b_ext)

    out2 = pl.pallas_call(
        _attn_kernel,
        grid=(B, NT),
        in_specs=[
            pl.BlockSpec((S, HE), lambda b, i: (b, 0)),
            pl.BlockSpec((1, TQ, S), lambda b, i: (b, i, 0)),
            pl.BlockSpec((H, H), lambda b, i: (0, 0)),
            pl.BlockSpec((1, H), lambda b, i: (0, 0)),
        ],
        out_specs=pl.BlockSpec((TQ, H), lambda b, i: (b * NT + i, 0)),
        out_shape=jax.ShapeDtypeStruct((B * S, H), jnp.float32),
    )(mixed, mask3, w_dense16, b_dense.reshape(1, H))

    return out2.reshape(B, S, H).transpose(1, 0, 2)


# speedup vs baseline: 1.1618x; 1.0257x over previous
import jax
import jax.numpy as jnp
from jax.experimental import pallas as pl

S = 2048
B = 2
H = 768
NH = 12
HPH = 64
TQ = 256
NT = S // TQ
HB = 4 * HPH
HE = NH * HB
SCALE = 1.0 / (HPH ** 0.5)
NEG = -1e9


def _qkv_kernel(x_ref, w_ref, b_ref, o_ref):
    b = pl.program_id(0)
    x = x_ref[:, pl.ds(b * H, H)].astype(jnp.bfloat16)
    o_ref[...] = (
        jnp.dot(x, w_ref[...], preferred_element_type=jnp.float32)
        + b_ref[...]
    ).astype(jnp.bfloat16)


def _attn_kernel(kv_ref, mask_ref, wd_ref, bd_ref, o_ref):
    i = pl.program_id(1)
    bias = (mask_ref[0] - 1.0) * -NEG
    outs = []
    for h in range(NH):
        base = h * HB
        k = kv_ref[:, base:base + HPH]
        q = kv_ref[pl.ds(i * TQ, TQ), base + HPH:base + 2 * HPH] * jnp.bfloat16(SCALE)
        vext = kv_ref[:, base + 2 * HPH:base + HB]
        s = jax.lax.dot_general(
            q, k, (((1,), (1,)), ((), ())),
            preferred_element_type=jnp.float32,
        )
        p = jnp.exp((s + bias).astype(jnp.bfloat16))
        oe = jax.lax.dot_general(
            p, vext, (((1,), (0,)), ((), ())),
            preferred_element_type=jnp.float32,
        )
        l = oe[:, HPH:HPH + 1]
        outs.append(oe[:, :HPH] / jnp.where(l == 0.0, 1.0, l))
    rep = jnp.concatenate(outs, axis=1).astype(jnp.bfloat16)
    o_ref[...] = (
        jnp.dot(rep, wd_ref[...], preferred_element_type=jnp.float32)
        + bd_ref[...]
    )


def kernel(hidden_states, attention_mask, W_qkv, b_qkv, W_dense, b_dense):
    hs2 = hidden_states.reshape(S, B * H)
    mask3 = attention_mask.reshape(B, S, S)

    w4 = W_qkv.reshape(H, NH, 3, HPH)
    w_ext = jnp.concatenate(
        [w4[:, :, 1], w4[:, :, 0], w4[:, :, 2],
         jnp.zeros((H, NH, HPH), W_qkv.dtype)], axis=-1,
    ).reshape(H, HE).astype(jnp.bfloat16)
    b4 = b_qkv.reshape(NH, 3, HPH)
    ones_col = jnp.zeros((NH, HPH), b_qkv.dtype).at[:, 0].set(1.0)
    b_ext = jnp.concatenate(
        [b4[:, 1], b4[:, 0], b4[:, 2], ones_col], axis=-1,
    ).reshape(1, HE)
    w_dense16 = W_dense.astype(jnp.bfloat16)

    mixed = pl.pallas_call(
        _qkv_kernel,
        grid=(B, NT),
        in_specs=[
            pl.BlockSpec((TQ, B * H), lambda b, i: (i, 0)),
            pl.BlockSpec((H, HE), lambda b, i: (0, 0)),
            pl.BlockSpec((1, HE), lambda b, i: (0, 0)),
        ],
        out_specs=pl.BlockSpec((TQ, HE), lambda b, i: (b * NT + i, 0)),
        out_shape=jax.ShapeDtypeStruct((B * S, HE), jnp.bfloat16),
    )(hs2, w_ext, b_ext)

    out2 = pl.pallas_call(
        _attn_kernel,
        grid=(B, NT),
        in_specs=[
            pl.BlockSpec((S, HE), lambda b, i: (b, 0)),
            pl.BlockSpec((1, TQ, S), lambda b, i: (b, i, 0)),
            pl.BlockSpec((H, H), lambda b, i: (0, 0)),
            pl.BlockSpec((1, H), lambda b, i: (0, 0)),
        ],
        out_specs=pl.BlockSpec((TQ, H), lambda b, i: (b * NT + i, 0)),
        out_shape=jax.ShapeDtypeStruct((B * S, H), jnp.float32),
    )(mixed, mask3, w_dense16, b_dense.reshape(1, H))

    return out2.reshape(B, S, H).transpose(1, 0, 2)


# speedup vs baseline: 1.2186x; 1.0488x over previous
import jax
import jax.numpy as jnp
from jax.experimental import pallas as pl

S = 2048
B = 2
H = 768
NH = 12
HPH = 64
TQ = 512
NT = S // TQ
HB = 4 * HPH
HE = NH * HB
SCALE = 1.0 / (HPH ** 0.5)
NEG = -1e9


def _qkv_kernel(x_ref, w_ref, b_ref, o_ref):
    b = pl.program_id(0)
    x = x_ref[:, pl.ds(b * H, H)].astype(jnp.bfloat16)
    o_ref[...] = (
        jnp.dot(x, w_ref[...], preferred_element_type=jnp.float32)
        + b_ref[...]
    ).astype(jnp.bfloat16)


def _attn_kernel(kv_ref, mask_ref, wd_ref, bd_ref, o_ref):
    i = pl.program_id(1)
    bias = (mask_ref[0] - 1.0) * -NEG
    outs = []
    for h in range(NH):
        base = h * HB
        k = kv_ref[:, base:base + HPH]
        q = kv_ref[pl.ds(i * TQ, TQ), base + HPH:base + 2 * HPH] * jnp.bfloat16(SCALE)
        vext = kv_ref[:, base + 2 * HPH:base + HB]
        s = jax.lax.dot_general(
            q, k, (((1,), (1,)), ((), ())),
            preferred_element_type=jnp.float32,
        )
        p = jnp.exp((s + bias).astype(jnp.bfloat16))
        oe = jax.lax.dot_general(
            p, vext, (((1,), (0,)), ((), ())),
            preferred_element_type=jnp.float32,
        )
        l = oe[:, HPH:HPH + 1]
        outs.append(oe[:, :HPH] / jnp.where(l == 0.0, 1.0, l))
    rep = jnp.concatenate(outs, axis=1).astype(jnp.bfloat16)
    o_ref[...] = (
        jnp.dot(rep, wd_ref[...], preferred_element_type=jnp.float32)
        + bd_ref[...]
    )


def kernel(hidden_states, attention_mask, W_qkv, b_qkv, W_dense, b_dense):
    hs2 = hidden_states.reshape(S, B * H)
    mask3 = attention_mask.reshape(B, S, S)

    w4 = W_qkv.reshape(H, NH, 3, HPH)
    w_ext = jnp.concatenate(
        [w4[:, :, 1], w4[:, :, 0], w4[:, :, 2],
         jnp.zeros((H, NH, HPH), W_qkv.dtype)], axis=-1,
    ).reshape(H, HE).astype(jnp.bfloat16)
    b4 = b_qkv.reshape(NH, 3, HPH)
    ones_col = jnp.zeros((NH, HPH), b_qkv.dtype).at[:, 0].set(1.0)
    b_ext = jnp.concatenate(
        [b4[:, 1], b4[:, 0], b4[:, 2], ones_col], axis=-1,
    ).reshape(1, HE)
    w_dense16 = W_dense.astype(jnp.bfloat16)

    mixed = pl.pallas_call(
        _qkv_kernel,
        grid=(B, NT),
        in_specs=[
            pl.BlockSpec((TQ, B * H), lambda b, i: (i, 0)),
            pl.BlockSpec((H, HE), lambda b, i: (0, 0)),
            pl.BlockSpec((1, HE), lambda b, i: (0, 0)),
        ],
        out_specs=pl.BlockSpec((TQ, HE), lambda b, i: (b * NT + i, 0)),
        out_shape=jax.ShapeDtypeStruct((B * S, HE), jnp.bfloat16),
    )(hs2, w_ext, b_ext)

    out2 = pl.pallas_call(
        _attn_kernel,
        grid=(B, NT),
        in_specs=[
            pl.BlockSpec((S, HE), lambda b, i: (b, 0)),
            pl.BlockSpec((1, TQ, S), lambda b, i: (b, i, 0)),
            pl.BlockSpec((H, H), lambda b, i: (0, 0)),
            pl.BlockSpec((1, H), lambda b, i: (0, 0)),
        ],
        out_specs=pl.BlockSpec((TQ, H), lambda b, i: (b * NT + i, 0)),
        out_shape=jax.ShapeDtypeStruct((B * S, H), jnp.float32),
    )(mixed, mask3, w_dense16, b_dense.reshape(1, H))

    return out2.reshape(B, S, H).transpose(1, 0, 2)
